# baseline (speedup 1.0000x reference)
import functools

import jax
import jax.numpy as jnp
import numpy as np
from jax.experimental import pallas as pl

P_DROP = 0.5
EPS = 1e-8

_INT_MIN = np.int32(-2147483648)
_LOW31 = np.int32(0x7FFFFFFF)


def _f32_key(x):
    i = jax.lax.bitcast_convert_type(x, jnp.int32)
    return i ^ ((i >> 31) & _LOW31)


def _key_to_f32(key):
    i = key ^ ((key >> 31) & _LOW31)
    return jax.lax.bitcast_convert_type(i, jnp.float32)


def _select_rank(key, r):
    rows, cols = key.shape
    nchains = 32 if rows % (32 * 8) == 0 and cols % 128 == 0 else 1
    key3 = key.reshape(nchains, rows // nchains, cols)
    lo = jnp.full((1, cols), _INT_MIN, dtype=jnp.int32)
    for b in range(31, -1, -1):
        bit = _INT_MIN if b == 31 else jnp.int32(1 << b)
        mid = lo + bit
        ind = (key3 < mid[None]).astype(jnp.int32)
        c = jnp.sum(jnp.sum(ind, axis=1), axis=0, keepdims=True)
        lo = jnp.where(c <= r, mid, lo)
    return lo


def _cvar_kernel(x_ref, cvar_ref):
    x = x_ref[...]
    n = x.shape[0]
    inv_n = 1.0 / n
    s1 = jnp.sum(x, axis=0, keepdims=True)
    s2 = jnp.sum(x * x, axis=0, keepdims=True)
    var = s2 * inv_n - (s1 * inv_n) ** 2
    std = jnp.sqrt(jnp.maximum(var, 0.0))
    key = _f32_key(x)
    med_key = _select_rank(key, (n - 1) // 2)
    med = _key_to_f32(med_key)
    cvar_ref[...] = std / (jnp.abs(med) + EPS)


def _mask_kernel(cvar_ref, mask_ref, *, k, scale):
    cv = cvar_ref[...]
    d = cv.shape[1]
    key = _f32_key(cv)
    kt = key.reshape(d, 1)
    t = _select_rank(kt, d - k)
    greater = key > t
    g = jnp.sum(greater.astype(jnp.int32))
    quota = jnp.int32(k) - g
    eq = key == t
    tri = (jax.lax.broadcasted_iota(jnp.int32, (d, d), 0)
           < jax.lax.broadcasted_iota(jnp.int32, (d, d), 1)).astype(jnp.float32)
    pre = jnp.dot(eq.astype(jnp.float32), tri,
                  preferred_element_type=jnp.float32)
    drop = greater | (eq & (pre < quota.astype(jnp.float32)))
    mask_ref[...] = jnp.where(drop, 0.0, jnp.float32(scale))


def _apply_kernel(x_ref, mask_ref, o_ref):
    o_ref[...] = x_ref[...] * mask_ref[...]


def kernel(x):
    b, s, d = x.shape
    n = b * s
    k = max(1, int(round(d * P_DROP)))
    scale = 1.0 / (1.0 - k / float(d))
    x2 = x.reshape(n, d)

    cb = min(256, d)
    cvar = pl.pallas_call(
        _cvar_kernel,
        grid=(d // cb,),
        in_specs=[pl.BlockSpec((n, cb), lambda j: (0, j))],
        out_specs=pl.BlockSpec((1, cb), lambda j: (0, j)),
        out_shape=jax.ShapeDtypeStruct((1, d), jnp.float32),
    )(x2)

    mask = pl.pallas_call(
        functools.partial(_mask_kernel, k=k, scale=scale),
        in_specs=[pl.BlockSpec((1, d), lambda: (0, 0))],
        out_specs=pl.BlockSpec((1, d), lambda: (0, 0)),
        out_shape=jax.ShapeDtypeStruct((1, d), jnp.float32),
    )(cvar)

    rb = min(512, n)
    out2 = pl.pallas_call(
        _apply_kernel,
        grid=(n // rb,),
        in_specs=[pl.BlockSpec((rb, d), lambda i: (i, 0)),
                  pl.BlockSpec((1, d), lambda i: (0, 0))],
        out_specs=pl.BlockSpec((rb, d), lambda i: (i, 0)),
        out_shape=jax.ShapeDtypeStruct((n, d), jnp.float32),
    )(x2, mask)
    return out2.reshape(b, s, d)


# speedup vs baseline: 1.1309x; 1.1309x over previous
import dataclasses
import functools

import jax
import jax.numpy as jnp
import numpy as np
from jax import lax
from jax.experimental import pallas as pl
from jax.experimental.pallas import tpu as pltpu
from jax.experimental.pallas import tpu_sc as plsc

P_DROP = 0.5
EPS = 1e-8

_INT_MIN = np.int32(-2147483648)
_LOW31 = np.int32(0x7FFFFFFF)


def _f32_key(x):
    i = jax.lax.bitcast_convert_type(x, jnp.int32)
    return i ^ ((i >> 31) & _LOW31)


def _key_to_f32(key):
    i = key ^ ((key >> 31) & _LOW31)
    return jax.lax.bitcast_convert_type(i, jnp.float32)


def _select_rank(key, r):
    rows, cols = key.shape
    nchains = 32 if rows % (32 * 8) == 0 and cols % 128 == 0 else 1
    key3 = key.reshape(nchains, rows // nchains, cols)
    lo = jnp.full((1, cols), _INT_MIN, dtype=jnp.int32)
    for b in range(31, -1, -1):
        bit = _INT_MIN if b == 31 else jnp.int32(1 << b)
        mid = lo + bit
        ind = (key3 < mid[None]).astype(jnp.int32)
        c = jnp.sum(jnp.sum(ind, axis=1), axis=0, keepdims=True)
        lo = jnp.where(c <= r, mid, lo)
    return lo


def _cvar_kernel(x_ref, cvar_ref):
    x = x_ref[...]
    n = x.shape[0]
    inv_n = 1.0 / n
    s1 = jnp.sum(x, axis=0, keepdims=True)
    s2 = jnp.sum(x * x, axis=0, keepdims=True)
    var = s2 * inv_n - (s1 * inv_n) ** 2
    std = jnp.sqrt(jnp.maximum(var, 0.0))
    key = _f32_key(x)
    med_key = _select_rank(key, (n - 1) // 2)
    med = _key_to_f32(med_key)
    cvar_ref[...] = std / (jnp.abs(med) + EPS)


def _mask_kernel(cvar_ref, mask_ref, *, k, scale):
    cv = cvar_ref[...]
    d = cv.shape[1]
    key = _f32_key(cv)
    kt = key.reshape(d, 1)
    t = _select_rank(kt, d - k)
    greater = key > t
    g = jnp.sum(greater.astype(jnp.int32))
    quota = jnp.int32(k) - g
    eq = key == t
    tri = (jax.lax.broadcasted_iota(jnp.int32, (d, d), 0)
           < jax.lax.broadcasted_iota(jnp.int32, (d, d), 1)).astype(jnp.float32)
    pre = jnp.dot(eq.astype(jnp.float32), tri,
                  preferred_element_type=jnp.float32)
    drop = greater | (eq & (pre < quota.astype(jnp.float32)))
    mask_ref[...] = jnp.where(drop, 0.0, jnp.float32(scale))


def _apply_kernel(x_ref, mask_ref, o_ref):
    o_ref[...] = x_ref[...] * mask_ref[...]


def _sc_mask(cvar_flat, *, k, scale):
    d = cvar_flat.shape[0]
    nchunks = d // 16
    mesh = plsc.VectorSubcoreMesh(core_axis_name="c", subcore_axis_name="s")
    cp = pltpu.CompilerParams()
    if "needs_layout_passes" in pltpu.CompilerParams.__dataclass_fields__:
        cp = dataclasses.replace(cp, needs_layout_passes=False)

    @functools.partial(
        pl.kernel, mesh=mesh, compiler_params=cp,
        out_type=jax.ShapeDtypeStruct((d,), jnp.float32),
        scratch_types=[
            pltpu.VMEM((d,), jnp.int32),
            pltpu.VMEM((d,), jnp.float32),
            pltpu.VMEM((16,), jnp.int32),
            pltpu.SMEM((4,), jnp.int32),
            pltpu.SemaphoreType.DMA,
        ],
    )
    def sc_kernel(cv_hbm, mask_hbm, key_v, mask_v, acc_v, s_ref, sem):
        wid = lax.axis_index("s") * 2 + lax.axis_index("c")

        @pl.when(wid == 0)
        def _():
            pltpu.async_copy(cv_hbm, mask_v, sem).wait()

            @pl.loop(0, nchunks)
            def _(c):
                v = mask_v[pl.ds(c * 16, 16)]
                i = lax.bitcast_convert_type(v, jnp.int32)
                key_v[pl.ds(c * 16, 16)] = i ^ ((i >> 31) & _LOW31)

            def count_less(pivot):
                acc_v[...] = jnp.zeros((16,), jnp.int32)

                @pl.loop(0, nchunks)
                def _(c):
                    kc = key_v[pl.ds(c * 16, 16)]
                    acc_v[...] += jnp.where(kc < pivot, 1, 0).astype(jnp.int32)

                return jnp.sum(acc_v[...])

            s_ref[0] = jnp.int32(_INT_MIN)
            r = d - k
            for b in range(31, -1, -1):
                bit = _INT_MIN if b == 31 else np.int32(1 << b)
                mid = s_ref[0] + bit
                cnt = count_less(mid)
                s_ref[0] = jnp.where(cnt <= r, mid, s_ref[0])

            t = s_ref[0]
            acc_v[...] = jnp.zeros((16,), jnp.int32)

            @pl.loop(0, nchunks)
            def _(c):
                kc = key_v[pl.ds(c * 16, 16)]
                acc_v[...] += jnp.where(kc > t, 1, 0).astype(jnp.int32)

            s_ref[2] = jnp.int32(k) - jnp.sum(acc_v[...])
            s_ref[1] = jnp.int32(0)

            @pl.loop(0, nchunks)
            def _(c):
                kc = key_v[pl.ds(c * 16, 16)]
                eq = kc == t
                eq_i = jnp.where(eq, 1, 0).astype(jnp.int32)
                excl = plsc.cumsum(eq_i) - eq_i
                gpref = excl + s_ref[1]
                drop = (kc > t) | (eq & (gpref < s_ref[2]))
                mask_v[pl.ds(c * 16, 16)] = jnp.where(
                    drop, jnp.float32(0.0), jnp.float32(scale))
                s_ref[1] = s_ref[1] + jnp.sum(eq_i)

            pltpu.async_copy(mask_v, mask_hbm, sem).wait()

    return sc_kernel(cvar_flat)


def kernel(x):
    b, s, d = x.shape
    n = b * s
    k = max(1, int(round(d * P_DROP)))
    scale = 1.0 / (1.0 - k / float(d))
    x2 = x.reshape(n, d)

    cb = min(128, d)
    cvar = pl.pallas_call(
        _cvar_kernel,
        grid=(d // cb,),
        in_specs=[pl.BlockSpec((n, cb), lambda j: (0, j))],
        out_specs=pl.BlockSpec((1, cb), lambda j: (0, j)),
        out_shape=jax.ShapeDtypeStruct((1, d), jnp.float32),
    )(x2)

    mask = _sc_mask(cvar.reshape(d), k=k, scale=scale).reshape(1, d)

    rb = min(512, n)
    out2 = pl.pallas_call(
        _apply_kernel,
        grid=(n // rb,),
        in_specs=[pl.BlockSpec((rb, d), lambda i: (i, 0)),
                  pl.BlockSpec((1, d), lambda i: (0, 0))],
        out_specs=pl.BlockSpec((rb, d), lambda i: (i, 0)),
        out_shape=jax.ShapeDtypeStruct((n, d), jnp.float32),
    )(x2, mask)
    return out2.reshape(b, s, d)


# speedup vs baseline: 1.3979x; 1.2362x over previous
import dataclasses
import functools

import jax
import jax.numpy as jnp
import numpy as np
from jax import lax
from jax.experimental import pallas as pl
from jax.experimental.pallas import tpu as pltpu
from jax.experimental.pallas import tpu_sc as plsc

P_DROP = 0.5
EPS = 1e-8

_INT_MIN = np.int32(-2147483648)
_LOW31 = np.int32(0x7FFFFFFF)


def _f32_key(x):
    i = jax.lax.bitcast_convert_type(x, jnp.int32)
    return i ^ ((i >> 31) & _LOW31)


def _key_to_f32(key):
    i = key ^ ((key >> 31) & _LOW31)
    return jax.lax.bitcast_convert_type(i, jnp.float32)


def _select_rank(key, r):
    rows, cols = key.shape
    if rows % 256 == 0 and cols % 128 == 0:
        ones = jnp.ones((1, rows), dtype=jnp.bfloat16)
        rf = jnp.float32(r)
        lo = jnp.full((1, cols), _INT_MIN, dtype=jnp.int32)
        for b in range(31, -1, -1):
            bit = _INT_MIN if b == 31 else jnp.int32(1 << b)
            mid = lo + bit
            ind = (key < mid).astype(jnp.bfloat16)
            c = jax.lax.dot_general(ones, ind, (((1,), (0,)), ((), ())),
                                    preferred_element_type=jnp.float32)
            lo = jnp.where(c <= rf, mid, lo)
        return lo
    nchains = 32 if rows % (32 * 8) == 0 and cols % 128 == 0 else 1
    key3 = key.reshape(nchains, rows // nchains, cols)
    lo = jnp.full((1, cols), _INT_MIN, dtype=jnp.int32)
    for b in range(31, -1, -1):
        bit = _INT_MIN if b == 31 else jnp.int32(1 << b)
        mid = lo + bit
        ind = (key3 < mid[None]).astype(jnp.int32)
        c = jnp.sum(jnp.sum(ind, axis=1), axis=0, keepdims=True)
        lo = jnp.where(c <= r, mid, lo)
    return lo


def _cvar_kernel(x_ref, cvar_ref):
    x = x_ref[...]
    n = x.shape[0]
    inv_n = 1.0 / n
    s1 = jnp.sum(x, axis=0, keepdims=True)
    s2 = jnp.sum(x * x, axis=0, keepdims=True)
    var = s2 * inv_n - (s1 * inv_n) ** 2
    std = jnp.sqrt(jnp.maximum(var, 0.0))
    key = _f32_key(x)
    med_key = _select_rank(key, (n - 1) // 2)
    med = _key_to_f32(med_key)
    cvar_ref[...] = std / (jnp.abs(med) + EPS)


def _mask_kernel(cvar_ref, mask_ref, *, k, scale):
    cv = cvar_ref[...]
    d = cv.shape[1]
    key = _f32_key(cv)
    kt = key.reshape(d, 1)
    t = _select_rank(kt, d - k)
    greater = key > t
    g = jnp.sum(greater.astype(jnp.int32))
    quota = jnp.int32(k) - g
    eq = key == t
    tri = (jax.lax.broadcasted_iota(jnp.int32, (d, d), 0)
           < jax.lax.broadcasted_iota(jnp.int32, (d, d), 1)).astype(jnp.float32)
    pre = jnp.dot(eq.astype(jnp.float32), tri,
                  preferred_element_type=jnp.float32)
    drop = greater | (eq & (pre < quota.astype(jnp.float32)))
    mask_ref[...] = jnp.where(drop, 0.0, jnp.float32(scale))


def _apply_kernel(x_ref, mask_ref, o_ref):
    o_ref[...] = x_ref[...] * mask_ref[...]


def _sc_mask(cvar_flat, *, k, scale):
    d = cvar_flat.shape[0]
    nchunks = d // 16
    mesh = plsc.VectorSubcoreMesh(core_axis_name="c", subcore_axis_name="s")
    cp = pltpu.CompilerParams()
    if "needs_layout_passes" in pltpu.CompilerParams.__dataclass_fields__:
        cp = dataclasses.replace(cp, needs_layout_passes=False)

    @functools.partial(
        pl.kernel, mesh=mesh, compiler_params=cp,
        out_type=jax.ShapeDtypeStruct((d,), jnp.float32),
        scratch_types=[
            pltpu.VMEM((d,), jnp.int32),
            pltpu.VMEM((d,), jnp.float32),
            pltpu.VMEM((16,), jnp.int32),
            pltpu.SMEM((4,), jnp.int32),
            pltpu.SemaphoreType.DMA,
        ],
    )
    def sc_kernel(cv_hbm, mask_hbm, key_v, mask_v, acc_v, s_ref, sem):
        wid = lax.axis_index("s") * 2 + lax.axis_index("c")

        @pl.when(wid == 0)
        def _():
            pltpu.async_copy(cv_hbm, mask_v, sem).wait()

            @pl.loop(0, nchunks)
            def _(c):
                v = mask_v[pl.ds(c * 16, 16)]
                i = lax.bitcast_convert_type(v, jnp.int32)
                key_v[pl.ds(c * 16, 16)] = i ^ ((i >> 31) & _LOW31)

            def count_less(pivot):
                acc_v[...] = jnp.zeros((16,), jnp.int32)

                @pl.loop(0, nchunks)
                def _(c):
                    kc = key_v[pl.ds(c * 16, 16)]
                    acc_v[...] += jnp.where(kc < pivot, 1, 0).astype(jnp.int32)

                return jnp.sum(acc_v[...])

            s_ref[0] = jnp.int32(_INT_MIN)
            r = d - k
            for b in range(31, -1, -1):
                bit = _INT_MIN if b == 31 else np.int32(1 << b)
                mid = s_ref[0] + bit
                cnt = count_less(mid)
                s_ref[0] = jnp.where(cnt <= r, mid, s_ref[0])

            t = s_ref[0]
            acc_v[...] = jnp.zeros((16,), jnp.int32)

            @pl.loop(0, nchunks)
            def _(c):
                kc = key_v[pl.ds(c * 16, 16)]
                acc_v[...] += jnp.where(kc > t, 1, 0).astype(jnp.int32)

            s_ref[2] = jnp.int32(k) - jnp.sum(acc_v[...])
            s_ref[1] = jnp.int32(0)

            @pl.loop(0, nchunks)
            def _(c):
                kc = key_v[pl.ds(c * 16, 16)]
                eq = kc == t
                eq_i = jnp.where(eq, 1, 0).astype(jnp.int32)
                excl = plsc.cumsum(eq_i) - eq_i
                gpref = excl + s_ref[1]
                drop = (kc > t) | (eq & (gpref < s_ref[2]))
                mask_v[pl.ds(c * 16, 16)] = jnp.where(
                    drop, jnp.float32(0.0), jnp.float32(scale))
                s_ref[1] = s_ref[1] + jnp.sum(eq_i)

            pltpu.async_copy(mask_v, mask_hbm, sem).wait()

    return sc_kernel(cvar_flat)


def kernel(x):
    b, s, d = x.shape
    n = b * s
    k = max(1, int(round(d * P_DROP)))
    scale = 1.0 / (1.0 - k / float(d))
    x2 = x.reshape(n, d)

    cb = min(128, d)
    cvar = pl.pallas_call(
        _cvar_kernel,
        grid=(d // cb,),
        in_specs=[pl.BlockSpec((n, cb), lambda j: (0, j))],
        out_specs=pl.BlockSpec((1, cb), lambda j: (0, j)),
        out_shape=jax.ShapeDtypeStruct((1, d), jnp.float32),
    )(x2)

    mask = _sc_mask(cvar.reshape(d), k=k, scale=scale).reshape(1, d)

    rb = min(512, n)
    out2 = pl.pallas_call(
        _apply_kernel,
        grid=(n // rb,),
        in_specs=[pl.BlockSpec((rb, d), lambda i: (i, 0)),
                  pl.BlockSpec((1, d), lambda i: (0, 0))],
        out_specs=pl.BlockSpec((rb, d), lambda i: (i, 0)),
        out_shape=jax.ShapeDtypeStruct((n, d), jnp.float32),
    )(x2, mask)
    return out2.reshape(b, s, d)


# speedup vs baseline: 1.4034x; 1.0039x over previous
import dataclasses
import functools

import jax
import jax.numpy as jnp
import numpy as np
from jax import lax
from jax.experimental import pallas as pl
from jax.experimental.pallas import tpu as pltpu
from jax.experimental.pallas import tpu_sc as plsc

P_DROP = 0.5
EPS = 1e-8

_INT_MIN = np.int32(-2147483648)
_LOW31 = np.int32(0x7FFFFFFF)


def _f32_key(x):
    i = jax.lax.bitcast_convert_type(x, jnp.int32)
    return i ^ ((i >> 31) & _LOW31)


def _key_to_f32(key):
    i = key ^ ((key >> 31) & _LOW31)
    return jax.lax.bitcast_convert_type(i, jnp.float32)


def _select_rank(key, r):
    rows, cols = key.shape
    if rows % 256 == 0 and cols % 128 == 0:
        ones = jnp.ones((1, rows), dtype=jnp.bfloat16)
        rf = jnp.float32(r)
        lo = jnp.full((1, cols), _INT_MIN, dtype=jnp.int32)
        for b in range(31, -1, -1):
            bit = _INT_MIN if b == 31 else jnp.int32(1 << b)
            mid = lo + bit
            ind = (key < mid).astype(jnp.bfloat16)
            c = jax.lax.dot_general(ones, ind, (((1,), (0,)), ((), ())),
                                    preferred_element_type=jnp.float32)
            lo = jnp.where(c <= rf, mid, lo)
        return lo
    nchains = 32 if rows % (32 * 8) == 0 and cols % 128 == 0 else 1
    key3 = key.reshape(nchains, rows // nchains, cols)
    lo = jnp.full((1, cols), _INT_MIN, dtype=jnp.int32)
    for b in range(31, -1, -1):
        bit = _INT_MIN if b == 31 else jnp.int32(1 << b)
        mid = lo + bit
        ind = (key3 < mid[None]).astype(jnp.int32)
        c = jnp.sum(jnp.sum(ind, axis=1), axis=0, keepdims=True)
        lo = jnp.where(c <= r, mid, lo)
    return lo


def _cvar_kernel(x_ref, cvar_ref):
    x = x_ref[...]
    n = x.shape[0]
    inv_n = 1.0 / n
    s1 = jnp.sum(x, axis=0, keepdims=True)
    s2 = jnp.sum(x * x, axis=0, keepdims=True)
    var = s2 * inv_n - (s1 * inv_n) ** 2
    std = jnp.sqrt(jnp.maximum(var, 0.0))
    key = _f32_key(x)
    med_key = _select_rank(key, (n - 1) // 2)
    med = _key_to_f32(med_key)
    cvar_ref[...] = std / (jnp.abs(med) + EPS)


def _apply_kernel(x_ref, mask_ref, o_ref):
    o_ref[...] = x_ref[...] * mask_ref[...]


def _sc_mask(cvar_flat, *, k, scale):
    d = cvar_flat.shape[0]
    nchunks = d // 16
    mesh = plsc.VectorSubcoreMesh(core_axis_name="c", subcore_axis_name="s")
    cp = pltpu.CompilerParams()
    if "needs_layout_passes" in pltpu.CompilerParams.__dataclass_fields__:
        cp = dataclasses.replace(cp, needs_layout_passes=False)

    @functools.partial(
        pl.kernel, mesh=mesh, compiler_params=cp,
        out_type=jax.ShapeDtypeStruct((d,), jnp.float32),
        scratch_types=[
            pltpu.VMEM((d,), jnp.int32),
            pltpu.VMEM((d,), jnp.float32),
            pltpu.VMEM((16,), jnp.int32),
            pltpu.SMEM((4,), jnp.int32),
            pltpu.SemaphoreType.DMA,
        ],
    )
    def sc_kernel(cv_hbm, mask_hbm, key_v, mask_v, acc_v, s_ref, sem):
        wid = lax.axis_index("s") * 2 + lax.axis_index("c")

        @pl.when(wid == 0)
        def _():
            pltpu.async_copy(cv_hbm, mask_v, sem).wait()

            @pl.loop(0, nchunks)
            def _(c):
                v = mask_v[pl.ds(c * 16, 16)]
                i = lax.bitcast_convert_type(v, jnp.int32)
                key_v[pl.ds(c * 16, 16)] = i ^ ((i >> 31) & _LOW31)

            def count_less(pivot):
                acc_v[...] = jnp.zeros((16,), jnp.int32)

                @pl.loop(0, nchunks)
                def _(c):
                    kc = key_v[pl.ds(c * 16, 16)]
                    acc_v[...] += jnp.where(kc < pivot, 1, 0).astype(jnp.int32)

                return jnp.sum(acc_v[...])

            s_ref[0] = jnp.int32(_INT_MIN)
            r = d - k
            for b in range(31, -1, -1):
                bit = _INT_MIN if b == 31 else np.int32(1 << b)
                mid = s_ref[0] + bit
                cnt = count_less(mid)
                s_ref[0] = jnp.where(cnt <= r, mid, s_ref[0])

            t = s_ref[0]
            acc_v[...] = jnp.zeros((16,), jnp.int32)

            @pl.loop(0, nchunks)
            def _(c):
                kc = key_v[pl.ds(c * 16, 16)]
                acc_v[...] += jnp.where(kc > t, 1, 0).astype(jnp.int32)

            s_ref[2] = jnp.int32(k) - jnp.sum(acc_v[...])
            s_ref[1] = jnp.int32(0)

            @pl.loop(0, nchunks)
            def _(c):
                kc = key_v[pl.ds(c * 16, 16)]
                eq = kc == t
                eq_i = jnp.where(eq, 1, 0).astype(jnp.int32)
                excl = plsc.cumsum(eq_i) - eq_i
                gpref = excl + s_ref[1]
                drop = (kc > t) | (eq & (gpref < s_ref[2]))
                mask_v[pl.ds(c * 16, 16)] = jnp.where(
                    drop, jnp.float32(0.0), jnp.float32(scale))
                s_ref[1] = s_ref[1] + jnp.sum(eq_i)

            pltpu.async_copy(mask_v, mask_hbm, sem).wait()

    return sc_kernel(cvar_flat)


def kernel(x):
    b, s, d = x.shape
    n = b * s
    k = max(1, int(round(d * P_DROP)))
    scale = 1.0 / (1.0 - k / float(d))
    x2 = x.reshape(n, d)

    cb = min(128, d)
    cvar = pl.pallas_call(
        _cvar_kernel,
        grid=(d // cb,),
        in_specs=[pl.BlockSpec((n, cb), lambda j: (0, j))],
        out_specs=pl.BlockSpec((1, cb), lambda j: (0, j)),
        out_shape=jax.ShapeDtypeStruct((1, d), jnp.float32),
    )(x2)

    mask = _sc_mask(cvar.reshape(d), k=k, scale=scale).reshape(1, d)

    rb = min(1024, n)
    out2 = pl.pallas_call(
        _apply_kernel,
        grid=(n // rb,),
        in_specs=[pl.BlockSpec((rb, d), lambda i: (i, 0)),
                  pl.BlockSpec((1, d), lambda i: (0, 0))],
        out_specs=pl.BlockSpec((rb, d), lambda i: (i, 0)),
        out_shape=jax.ShapeDtypeStruct((n, d), jnp.float32),
    )(x2, mask)
    return out2.reshape(b, s, d)
